# baseline (speedup 1.0000x reference)
import functools

import jax
import jax.numpy as jnp
from jax import lax
from jax.experimental import pallas as pl
from jax.experimental.pallas import tpu as pltpu
from jax.experimental.pallas import tpu_sc as plsc

G = 2048
N_PTS = 1_000_000
C = 2048
D = 128
ND = C // D
NC, NS = 2, 16
NW = NC * NS
NCHUNK = (N_PTS + C - 1) // C
K_SCAT = (NCHUNK + NS - 1) // NS
K_GATH = (NCHUNK + NW - 1) // NW
GRID_W = (G + 2) * G

QR = 512
SB_ROWS = QR + 2
SB_W = SB_ROWS * G
TRASH = SB_W
SPM_W = SB_W + G
ZB = 16480
ZPW = SPM_W // NS
NZD = ZPW // ZB

assert ZPW * NS == SPM_W and NZD * ZB == ZPW

_mesh = plsc.VectorSubcoreMesh(
    core_axis_name="c", subcore_axis_name="s", num_cores=NC, num_subcores=NS
)


@functools.partial(
    pl.kernel,
    out_type=(
        jax.ShapeDtypeStruct((GRID_W,), jnp.float32),
        jax.ShapeDtypeStruct((N_PTS,), jnp.float32),
    ),
    mesh=_mesh,
    scratch_types=[
        pltpu.VMEM_SHARED((SPM_W,), jnp.float32),
        pltpu.VMEM((ZB,), jnp.float32),
        pltpu.VMEM((C,), jnp.int32),
        pltpu.VMEM((C,), jnp.int32),
        pltpu.VMEM((C,), jnp.float32),
        pltpu.VMEM((C,), jnp.float32),
        pltpu.VMEM((ND, D), jnp.int32),
        pltpu.VMEM((ND, D), jnp.int32),
        pltpu.VMEM((ND, D), jnp.int32),
        pltpu.VMEM((ND, D), jnp.int32),
        pltpu.VMEM((C,), jnp.float32),
        pltpu.VMEM((C,), jnp.float32),
        pltpu.VMEM((C,), jnp.float32),
        pltpu.VMEM((C,), jnp.float32),
        pltpu.SemaphoreType.DMA,
        pltpu.SemaphoreType.DMA,
        pltpu.SemaphoreType.DMA,
        pltpu.SemaphoreType.DMA,
        pltpu.SemaphoreType.REGULAR,
    ],
    name="sc_fused",
)
def _fused(lin_hbm, f_hbm, grid_hbm, out_hbm,
           spm, zb, linA, linB, fA, fB, idxA, idxB, idxMA, idxMB,
           gMA, gMB, obA, obB, siA, siB, sgA, sgB, bsem):
  c = lax.axis_index("c")
  s = lax.axis_index("s")

  def zvec(j, carry):
    zb[pl.ds(j * 16, 16)] = jnp.zeros((16,), jnp.float32)
    return carry
  lax.fori_loop(0, ZB // 16, zvec, 0, unroll=4)

  def s_valid(t):
    return (t >= 0) & (s + NS * t < NCHUNK)

  def s_base(t):
    return jnp.minimum((s + NS * t) * C, N_PTS - C)

  def s_firein(t, lin_, f_, si_):
    @pl.when(s_valid(t))
    def _():
      b = s_base(t)
      pltpu.make_async_copy(lin_hbm.at[pl.ds(b, C)], lin_, si_).start()
      pltpu.make_async_copy(f_hbm.at[pl.ds(b, C)], f_, si_).start()

  def s_waitin(t, lin_, f_, si_):
    @pl.when(s_valid(t))
    def _():
      b = s_base(t)
      pltpu.make_async_copy(lin_hbm.at[pl.ds(b, C)], lin_, si_).wait()
      pltpu.make_async_copy(f_hbm.at[pl.ds(b, C)], f_, si_).wait()

  def s_compute(t, lin_, idx_, soff):
    @pl.when(s_valid(t))
    def _():
      def vec(j, c2):
        lv = lin_[pl.ds(j * 16, 16)] + soff
        owned = (lv >= G) & (lv < (QR + 1) * G)
        idx_[j // 8, pl.ds((j % 8) * 16, 16)] = jnp.where(
            owned, lv, TRASH + (lv & (G - 1)))
        return c2
      lax.fori_loop(0, C // 16, vec, 0, unroll=4)

  def s_fire(t, f_, idx_, sg_):
    @pl.when(s_valid(t))
    def _():
      for d in range(ND):
        pltpu.make_async_copy(
            f_.at[pl.ds(d * D, D)], spm.at[idx_.at[d]], sg_).start()

  def s_drain(t, f_, idx_, sg_):
    @pl.when(s_valid(t))
    def _():
      for d in range(ND):
        pltpu.make_async_copy(
            f_.at[pl.ds(d * D, D)], spm.at[idx_.at[d]], sg_).wait()

  sbufs = ((linA, fA, idxA, siA, sgA), (linB, fB, idxB, siB, sgB))

  for p in range(2):
    qbase = 1024 * p + 512 * c
    soff = (1 - qbase) * G

    for k in range(NZD):
      pltpu.sync_copy(zb, spm.at[pl.ds(s * ZPW + k * ZB, ZB)])
    plsc.subcore_barrier()

    s_firein(0, linA, fA, siA)

    def s_double(u, carry):
      for par in range(2):
        t = 2 * u + par
        lin_, f_, idx_, si_, sg_ = sbufs[par]
        linO, fO, idxO, siO, sgO = sbufs[1 - par]
        s_waitin(t, lin_, f_, si_)
        s_compute(t, lin_, idx_, soff)
        s_drain(t - 1, fO, idxO, sgO)
        s_fire(t, f_, idx_, sg_)
        s_firein(t + 1, linO, fO, siO)
      return carry

    lax.fori_loop(0, (K_SCAT + 2) // 2, s_double, 0)
    plsc.subcore_barrier()

    row0 = 1 + 32 * s
    pltpu.sync_copy(
        spm.at[pl.ds(row0 * G, 32 * G)],
        grid_hbm.at[pl.ds((qbase + row0) * G, 32 * G)],
    )
    if p == 0:
      @pl.when((c == 0) & (s == 0))
      def _():
        pltpu.sync_copy(spm.at[pl.ds(0, G)], grid_hbm.at[pl.ds(0, G)])
    else:
      @pl.when((c == 1) & (s == 0))
      def _():
        pltpu.sync_copy(
            spm.at[pl.ds((QR + 1) * G, G)],
            grid_hbm.at[pl.ds((G + 1) * G, G)],
        )
    plsc.subcore_barrier()

  pltpu.core_barrier(bsem, core_axis_name="c")

  wid = s * NC + c

  def g_valid(t):
    return (t >= 0) & (wid + NW * t < NCHUNK)

  def g_base(t):
    return jnp.minimum((wid + NW * t) * C, N_PTS - C)

  def g_firein(t, lin_, si_):
    @pl.when(g_valid(t))
    def _():
      pltpu.make_async_copy(
          lin_hbm.at[pl.ds(g_base(t), C)], lin_, si_).start()

  def g_waitin(t, lin_, si_):
    @pl.when(g_valid(t))
    def _():
      pltpu.make_async_copy(
          lin_hbm.at[pl.ds(g_base(t), C)], lin_, si_).wait()

  def g_compute(t, lin_, idxP_, idxM_):
    @pl.when(g_valid(t))
    def _():
      def vec(j, c2):
        lv = lin_[pl.ds(j * 16, 16)]
        idxP_[j // 8, pl.ds((j % 8) * 16, 16)] = lv + 2 * G
        idxM_[j // 8, pl.ds((j % 8) * 16, 16)] = lv
        return c2
      lax.fori_loop(0, C // 16, vec, 0, unroll=4)

  def g_fire(t, idxP_, idxM_, gP_, gM_, sg_):
    @pl.when(g_valid(t))
    def _():
      for d in range(ND):
        pltpu.make_async_copy(
            grid_hbm.at[idxP_.at[d]], gP_.at[pl.ds(d * D, D)], sg_).start()
      for d in range(ND):
        pltpu.make_async_copy(
            grid_hbm.at[idxM_.at[d]], gM_.at[pl.ds(d * D, D)], sg_).start()

  def g_drain(t, idxP_, idxM_, gP_, gM_, sg_):
    @pl.when(g_valid(t))
    def _():
      for d in range(ND):
        pltpu.make_async_copy(
            grid_hbm.at[idxP_.at[d]], gP_.at[pl.ds(d * D, D)], sg_).wait()
      for d in range(ND):
        pltpu.make_async_copy(
            grid_hbm.at[idxM_.at[d]], gM_.at[pl.ds(d * D, D)], sg_).wait()

  def g_out(t, gP_, gM_, ob_):
    @pl.when(g_valid(t))
    def _():
      def ovec(j, c2):
        gp = gP_[pl.ds(j * 16, 16)]
        gm = gM_[pl.ds(j * 16, 16)]
        ob_[pl.ds(j * 16, 16)] = 0.5 * (gp - gm)
        return c2
      lax.fori_loop(0, C // 16, ovec, 0, unroll=4)
      pltpu.sync_copy(ob_, out_hbm.at[pl.ds(g_base(t), C)])

  gbufs = (
      (linA, idxA, idxMA, fA, gMA, obA, siA, sgA),
      (linB, idxB, idxMB, fB, gMB, obB, siB, sgB),
  )
  g_firein(0, linA, siA)

  def g_double(u, carry):
    for par in range(2):
      t = 2 * u + par
      lin_, idxP_, idxM_, gP_, gM_, ob_, si_, sg_ = gbufs[par]
      linO, idxPO, idxMO, gPO, gMO, obO, siO, sgO = gbufs[1 - par]
      g_waitin(t, lin_, si_)
      g_compute(t, lin_, idxP_, idxM_)
      g_drain(t - 1, idxPO, idxMO, gPO, gMO, sgO)
      g_fire(t, idxP_, idxM_, gP_, gM_, sg_)
      g_out(t - 1, gPO, gMO, obO)
      g_firein(t + 1, linO, siO)
    return carry

  lax.fori_loop(0, (K_GATH + 2) // 2, g_double, 0)


def kernel(feats, coords):
  lin = (coords[:, 0] * G + coords[:, 1]).astype(jnp.int32)
  f = feats[:, 0]
  _, out = _fused(lin, f)
  return out[:, None]


# speedup vs baseline: 1.0816x; 1.0816x over previous
import functools

import jax
import jax.numpy as jnp
from jax import lax
from jax.experimental import pallas as pl
from jax.experimental.pallas import tpu as pltpu
from jax.experimental.pallas import tpu_sc as plsc

G = 2048
N_PTS = 1_000_000
C = 2048
D = 128
ND = C // D
NC, NS = 2, 16
NW = NC * NS
NCHUNK = (N_PTS + C - 1) // C
K_SCAT = (NCHUNK + NS - 1) // NS
K_GATH = (NCHUNK + NW - 1) // NW
GRID_W = (G + 2) * G

QR = 512
SB_ROWS = QR + 2
SB_W = SB_ROWS * G
TRASH = SB_W
SPM_W = SB_W + G
ZB = 16480
ZPW = SPM_W // NS
NZD = ZPW // ZB

assert ZPW * NS == SPM_W and NZD * ZB == ZPW

_mesh = plsc.VectorSubcoreMesh(
    core_axis_name="c", subcore_axis_name="s", num_cores=NC, num_subcores=NS
)


@functools.partial(
    pl.kernel,
    out_type=(
        jax.ShapeDtypeStruct((GRID_W,), jnp.float32),
        jax.ShapeDtypeStruct((N_PTS,), jnp.float32),
    ),
    mesh=_mesh,
    scratch_types=[
        pltpu.VMEM_SHARED((SPM_W,), jnp.float32),
        pltpu.VMEM((ZB,), jnp.float32),
        pltpu.VMEM((C,), jnp.int32),
        pltpu.VMEM((C,), jnp.int32),
        pltpu.VMEM((C,), jnp.float32),
        pltpu.VMEM((C,), jnp.float32),
        pltpu.VMEM((ND, D), jnp.int32),
        pltpu.VMEM((ND, D), jnp.int32),
        pltpu.VMEM((ND, D), jnp.int32),
        pltpu.VMEM((ND, D), jnp.int32),
        pltpu.VMEM((C,), jnp.float32),
        pltpu.VMEM((C,), jnp.float32),
        pltpu.VMEM((C,), jnp.float32),
        pltpu.VMEM((C,), jnp.float32),
        pltpu.SemaphoreType.DMA,
        pltpu.SemaphoreType.DMA,
        pltpu.SemaphoreType.DMA,
        pltpu.SemaphoreType.DMA,
        pltpu.SemaphoreType.REGULAR,
    ],
    name="sc_fused",
)
def _fused(lin_hbm, f_hbm, grid_hbm, out_hbm,
           spm, zb, linA, linB, fA, fB, idxA, idxB, idxMA, idxMB,
           gMA, gMB, obA, obB, siA, siB, sgA, sgB, bsem):
  c = lax.axis_index("c")
  s = lax.axis_index("s")

  def zvec(j, carry):
    zb[pl.ds(j * 16, 16)] = jnp.zeros((16,), jnp.float32)
    return carry
  lax.fori_loop(0, ZB // 16, zvec, 0, unroll=4)

  def s_valid(t):
    return (t >= 0) & (s + NS * t < NCHUNK)

  def s_base(t):
    return jnp.minimum((s + NS * t) * C, N_PTS - C)

  def s_firein(t, lin_, f_, si_):
    @pl.when(s_valid(t))
    def _():
      b = s_base(t)
      pltpu.make_async_copy(lin_hbm.at[pl.ds(b, C)], lin_, si_).start()
      pltpu.make_async_copy(f_hbm.at[pl.ds(b, C)], f_, si_).start()

  def s_waitin(t, lin_, f_, si_):
    @pl.when(s_valid(t))
    def _():
      b = s_base(t)
      pltpu.make_async_copy(lin_hbm.at[pl.ds(b, C)], lin_, si_).wait()
      pltpu.make_async_copy(f_hbm.at[pl.ds(b, C)], f_, si_).wait()

  def s_compute(t, lin_, idx_, soff):
    @pl.when(s_valid(t))
    def _():
      def vec(j, c2):
        lv = lin_[pl.ds(j * 16, 16)] + soff
        owned = (lv >= G) & (lv < (QR + 1) * G)
        idx_[j // 8, pl.ds((j % 8) * 16, 16)] = jnp.where(
            owned, lv, TRASH + (lv & (G - 1)))
        return c2
      lax.fori_loop(0, C // 16, vec, 0, unroll=4)

  def s_fire(t, f_, idx_, sg_):
    @pl.when(s_valid(t))
    def _():
      for d in range(ND):
        pltpu.make_async_copy(
            f_.at[pl.ds(d * D, D)], spm.at[idx_.at[d]], sg_).start()

  def s_drain(t, f_, idx_, sg_):
    @pl.when(s_valid(t))
    def _():
      for d in range(ND):
        pltpu.make_async_copy(
            f_.at[pl.ds(d * D, D)], spm.at[idx_.at[d]], sg_).wait()

  sbufs = ((linA, fA, idxA, siA, sgA), (linB, fB, idxB, siB, sgB))

  for p in range(2):
    qbase = 1024 * p + 512 * c
    soff = (1 - qbase) * G

    s_firein(0, linA, fA, siA)
    for k in range(NZD):
      pltpu.sync_copy(zb, spm.at[pl.ds(s * ZPW + k * ZB, ZB)])
    plsc.subcore_barrier()

    def s_double(u, carry):
      for par in range(2):
        t = 2 * u + par
        lin_, f_, idx_, si_, sg_ = sbufs[par]
        linO, fO, idxO, siO, sgO = sbufs[1 - par]
        s_waitin(t, lin_, f_, si_)
        s_compute(t, lin_, idx_, soff)
        s_drain(t - 1, fO, idxO, sgO)
        s_fire(t, f_, idx_, sg_)
        s_firein(t + 1, linO, fO, siO)
      return carry

    lax.fori_loop(0, (K_SCAT + 2) // 2, s_double, 0)
    plsc.subcore_barrier()

    row0 = 1 + 32 * s
    pltpu.sync_copy(
        spm.at[pl.ds(row0 * G, 32 * G)],
        grid_hbm.at[pl.ds((qbase + row0) * G, 32 * G)],
    )
    if p == 0:
      @pl.when((c == 0) & (s == 0))
      def _():
        pltpu.sync_copy(spm.at[pl.ds(0, G)], grid_hbm.at[pl.ds(0, G)])
    else:
      @pl.when((c == 1) & (s == 0))
      def _():
        pltpu.sync_copy(
            spm.at[pl.ds((QR + 1) * G, G)],
            grid_hbm.at[pl.ds((G + 1) * G, G)],
        )
    plsc.subcore_barrier()

  pltpu.core_barrier(bsem, core_axis_name="c")

  wid = s * NC + c

  def g_valid(t):
    return (t >= 0) & (wid + NW * t < NCHUNK)

  def g_base(t):
    return jnp.minimum((wid + NW * t) * C, N_PTS - C)

  def g_firein(t, lin_, si_):
    @pl.when(g_valid(t))
    def _():
      pltpu.make_async_copy(
          lin_hbm.at[pl.ds(g_base(t), C)], lin_, si_).start()

  def g_waitin(t, lin_, si_):
    @pl.when(g_valid(t))
    def _():
      pltpu.make_async_copy(
          lin_hbm.at[pl.ds(g_base(t), C)], lin_, si_).wait()

  def g_compute(t, lin_, idxP_, idxM_):
    @pl.when(g_valid(t))
    def _():
      def vec(j, c2):
        lv = lin_[pl.ds(j * 16, 16)]
        idxP_[j // 8, pl.ds((j % 8) * 16, 16)] = lv + 2 * G
        idxM_[j // 8, pl.ds((j % 8) * 16, 16)] = lv
        return c2
      lax.fori_loop(0, C // 16, vec, 0, unroll=4)

  def g_fire(t, idxP_, idxM_, gP_, gM_, sg_):
    @pl.when(g_valid(t))
    def _():
      for d in range(ND):
        pltpu.make_async_copy(
            grid_hbm.at[idxP_.at[d]], gP_.at[pl.ds(d * D, D)], sg_).start()
      for d in range(ND):
        pltpu.make_async_copy(
            grid_hbm.at[idxM_.at[d]], gM_.at[pl.ds(d * D, D)], sg_).start()

  def g_drain(t, idxP_, idxM_, gP_, gM_, sg_):
    @pl.when(g_valid(t))
    def _():
      for d in range(ND):
        pltpu.make_async_copy(
            grid_hbm.at[idxP_.at[d]], gP_.at[pl.ds(d * D, D)], sg_).wait()
      for d in range(ND):
        pltpu.make_async_copy(
            grid_hbm.at[idxM_.at[d]], gM_.at[pl.ds(d * D, D)], sg_).wait()

  def g_out(t, gP_, gM_, ob_):
    @pl.when(g_valid(t))
    def _():
      def ovec(j, c2):
        gp = gP_[pl.ds(j * 16, 16)]
        gm = gM_[pl.ds(j * 16, 16)]
        ob_[pl.ds(j * 16, 16)] = 0.5 * (gp - gm)
        return c2
      lax.fori_loop(0, C // 16, ovec, 0, unroll=4)
      pltpu.sync_copy(ob_, out_hbm.at[pl.ds(g_base(t), C)])

  gbufs = (
      (linA, idxA, idxMA, fA, gMA, obA, siA, sgA),
      (linB, idxB, idxMB, fB, gMB, obB, siB, sgB),
  )
  g_firein(0, linA, siA)

  def g_double(u, carry):
    for par in range(2):
      t = 2 * u + par
      lin_, idxP_, idxM_, gP_, gM_, ob_, si_, sg_ = gbufs[par]
      linO, idxPO, idxMO, gPO, gMO, obO, siO, sgO = gbufs[1 - par]
      g_waitin(t, lin_, si_)
      g_compute(t, lin_, idxP_, idxM_)
      g_drain(t - 1, idxPO, idxMO, gPO, gMO, sgO)
      g_fire(t, idxP_, idxM_, gP_, gM_, sg_)
      g_out(t - 1, gPO, gMO, obO)
      g_firein(t + 1, linO, siO)
    return carry

  lax.fori_loop(0, (K_GATH + 2) // 2, g_double, 0)


def kernel(feats, coords):
  lin = (coords[:, 0] * G + coords[:, 1]).astype(jnp.int32)
  f = feats[:, 0]
  _, out = _fused(lin, f)
  return out[:, None]
